# baseline (speedup 1.0000x reference)
import functools

import jax
import jax.numpy as jnp
from jax.experimental import pallas as pl
from jax.experimental.pallas import tpu as pltpu

_NUM_EXPERTS = 16
_BLK = 1024


def _gate_kernel(h_ref, wt_ref, b_ref, out_ref):
    g = jnp.dot(h_ref[...], wt_ref[...], preferred_element_type=jnp.float32)
    g = g + b_ref[...]
    idx = jax.lax.broadcasted_iota(jnp.int32, g.shape, 1)
    m1 = jnp.max(g, axis=1, keepdims=True)
    i1 = jnp.min(jnp.where(g == m1, idx, _NUM_EXPERTS), axis=1, keepdims=True)
    g2 = jnp.where(idx == i1, -jnp.inf, g)
    m2 = jnp.max(g2, axis=1, keepdims=True)
    i2 = jnp.min(jnp.where(g2 == m2, idx, _NUM_EXPERTS), axis=1, keepdims=True)
    mask = (idx == i1) | (idx == i2)
    e = jnp.exp(g - m1)
    denom = 1.0 + jnp.exp(m2 - m1)
    out_ref[...] = jnp.where(mask, e / denom, 0.0)


@functools.partial(jax.jit, static_argnames=())
def kernel(h, W, b):
    n, d = h.shape
    ne = W.shape[0]
    wt = W.T
    b2 = b.reshape(1, ne)
    grid = (n // _BLK,)
    return pl.pallas_call(
        _gate_kernel,
        grid=grid,
        in_specs=[
            pl.BlockSpec((_BLK, d), lambda i: (i, 0)),
            pl.BlockSpec((d, ne), lambda i: (0, 0)),
            pl.BlockSpec((1, ne), lambda i: (0, 0)),
        ],
        out_specs=pl.BlockSpec((_BLK, ne), lambda i: (i, 0)),
        out_shape=jax.ShapeDtypeStruct((n, ne), jnp.float32),
        compiler_params=pltpu.CompilerParams(
            dimension_semantics=("parallel",),
        ),
    )(h, wt, b2)


# speedup vs baseline: 1.0419x; 1.0419x over previous
import functools

import jax
import jax.numpy as jnp
from jax.experimental import pallas as pl
from jax.experimental.pallas import tpu as pltpu

_NUM_EXPERTS = 16
_BLK = 1024


def _gate_kernel(h_ref, wt_ref, b_ref, out_ref):
    g = jnp.dot(h_ref[...], wt_ref[...], preferred_element_type=jnp.float32)
    g = g + b_ref[...]
    idx = jax.lax.broadcasted_iota(jnp.int32, g.shape, 1).astype(jnp.float32)
    ne_f = jnp.float32(_NUM_EXPERTS)
    m1 = jnp.max(g, axis=1, keepdims=True)
    i1 = jnp.min(jnp.where(g == m1, idx, ne_f), axis=1, keepdims=True)
    g2 = jnp.where(idx == i1, -jnp.inf, g)
    m2 = jnp.max(g2, axis=1, keepdims=True)
    i2 = jnp.min(jnp.where(g2 == m2, idx, ne_f), axis=1, keepdims=True)
    mask = (idx == i1) | (idx == i2)
    e = jnp.exp(g - m1)
    denom = 1.0 + jnp.exp(m2 - m1)
    out_ref[...] = jnp.where(mask, e / denom, 0.0)


@functools.partial(jax.jit, static_argnames=())
def kernel(h, W, b):
    n, d = h.shape
    ne = W.shape[0]
    wt = W.T
    b2 = b.reshape(1, ne)
    grid = (n // _BLK,)
    return pl.pallas_call(
        _gate_kernel,
        grid=grid,
        in_specs=[
            pl.BlockSpec((_BLK, d), lambda i: (i, 0)),
            pl.BlockSpec((d, ne), lambda i: (0, 0)),
            pl.BlockSpec((1, ne), lambda i: (0, 0)),
        ],
        out_specs=pl.BlockSpec((_BLK, ne), lambda i: (i, 0)),
        out_shape=jax.ShapeDtypeStruct((n, ne), jnp.float32),
        compiler_params=pltpu.CompilerParams(
            dimension_semantics=("parallel",),
        ),
    )(h, wt, b2)


# speedup vs baseline: 1.0624x; 1.0197x over previous
import functools

import jax
import jax.numpy as jnp
from jax.experimental import pallas as pl
from jax.experimental.pallas import tpu as pltpu

_NUM_EXPERTS = 16
_BLK = 2048


def _gate_kernel(h_ref, wt_ref, b_ref, out_ref):
    g = jnp.dot(h_ref[...], wt_ref[...], preferred_element_type=jnp.float32)
    g = g + b_ref[...]
    idx = jax.lax.broadcasted_iota(jnp.int32, g.shape, 1).astype(jnp.float32)
    ne_f = jnp.float32(_NUM_EXPERTS)
    m1 = jnp.max(g, axis=1, keepdims=True)
    i1 = jnp.min(jnp.where(g == m1, idx, ne_f), axis=1, keepdims=True)
    g2 = jnp.where(idx == i1, -jnp.inf, g)
    m2 = jnp.max(g2, axis=1, keepdims=True)
    i2 = jnp.min(jnp.where(g2 == m2, idx, ne_f), axis=1, keepdims=True)
    mask = (idx == i1) | (idx == i2)
    e = jnp.exp(g - m1)
    denom = 1.0 + jnp.exp(m2 - m1)
    out_ref[...] = jnp.where(mask, e / denom, 0.0)


@functools.partial(jax.jit, static_argnames=())
def kernel(h, W, b):
    n, d = h.shape
    ne = W.shape[0]
    wt = W.T
    b2 = b.reshape(1, ne)
    grid = (n // _BLK,)
    return pl.pallas_call(
        _gate_kernel,
        grid=grid,
        in_specs=[
            pl.BlockSpec((_BLK, d), lambda i: (i, 0)),
            pl.BlockSpec((d, ne), lambda i: (0, 0)),
            pl.BlockSpec((1, ne), lambda i: (0, 0)),
        ],
        out_specs=pl.BlockSpec((_BLK, ne), lambda i: (i, 0)),
        out_shape=jax.ShapeDtypeStruct((n, ne), jnp.float32),
        compiler_params=pltpu.CompilerParams(
            dimension_semantics=("parallel",),
        ),
    )(h, wt, b2)
